# Initial kernel scaffold; baseline (speedup 1.0000x reference)
import jax
import jax.numpy as jnp
from jax.experimental import pallas as pl


def kernel(input_ids, table, W, b):
    raise NotImplementedError("write your pallas kernel here")


# speedup vs baseline: 2.3714x; 2.3714x over previous
import functools

import jax
import jax.numpy as jnp
from jax import lax
from jax.experimental import pallas as pl
from jax.experimental.pallas import tpu as pltpu
from jax.experimental.pallas import tpu_sc as plsc

VOCAB = 1000000
EMBED = 64
B = 16384
L = 50

NC = 2
NS = 16
NW = NC * NS
BPW = B // NW
CH = 32
IDS_PER_CHUNK = CH * L
G = 80
NG = IDS_PER_CHUNK // G
NCHUNK = BPW // CH

_sc_mesh = plsc.VectorSubcoreMesh(core_axis_name="c", subcore_axis_name="s")


@functools.partial(
    pl.kernel,
    mesh=_sc_mesh,
    out_type=jax.ShapeDtypeStruct((B, EMBED), jnp.float32),
    scratch_types=[
        pltpu.VMEM((IDS_PER_CHUNK,), jnp.int32),
        pltpu.VMEM((IDS_PER_CHUNK, EMBED), jnp.float32),
        pltpu.VMEM((CH, EMBED), jnp.float32),
        pltpu.SemaphoreType.DMA,
    ],
    compiler_params=pltpu.CompilerParams(use_tc_tiling_on_sc=False),
)
def _sc_pool(ids_hbm, table_hbm, out_hbm, ids_v, rows_v, pooled_v, sem):
    wid = lax.axis_index("s") * NC + lax.axis_index("c")

    def chunk_body(ci, carry):
        chunk = wid * NCHUNK + ci
        pltpu.sync_copy(ids_hbm.at[pl.ds(chunk * IDS_PER_CHUNK, IDS_PER_CHUNK)],
                        ids_v)
        descs = []
        for g in range(NG):
            descs.append(pltpu.async_copy(
                table_hbm.at[ids_v.at[pl.ds(g * G, G)]],
                rows_v.at[pl.ds(g * G, G)],
                sem,
            ))
        for d in descs:
            d.wait()

        def b_body(bi, c2):
            row0 = bi * L
            for col in range(EMBED // 16):
                acc = rows_v[row0, pl.ds(col * 16, 16)]
                for j in range(1, L):
                    acc = acc + rows_v[row0 + j, pl.ds(col * 16, 16)]
                pooled_v[bi, pl.ds(col * 16, 16)] = acc
            return c2

        lax.fori_loop(0, CH, b_body, 0, unroll=False)
        pltpu.sync_copy(pooled_v, out_hbm.at[pl.ds(chunk * CH, CH)])
        return carry

    lax.fori_loop(0, NCHUNK, chunk_body, 0, unroll=False)


def _tc_proj(x_ref, w_ref, b_ref, o_ref):
    x = x_ref[...] * (1.0 / L)
    y = jnp.dot(x, w_ref[...].T, preferred_element_type=jnp.float32)
    y = y + b_ref[...]
    n = jnp.sqrt(jnp.sum(y * y, axis=-1, keepdims=True))
    o_ref[...] = y / jnp.maximum(n, 1e-12)


def kernel(input_ids, table, W, b):
    flat_ids = input_ids.reshape(-1)
    pooled = _sc_pool(flat_ids, table)
    out = pl.pallas_call(
        _tc_proj,
        out_shape=jax.ShapeDtypeStruct((B, EMBED), jnp.float32),
    )(pooled, W, b.reshape(1, EMBED))
    return out
